# Initial kernel scaffold; baseline (speedup 1.0000x reference)
import jax
import jax.numpy as jnp
from jax.experimental import pallas as pl


def kernel(edge_index, edge_attr, pol_features, state_ids, comp_features, Wp, bp, state_emb, sector_emb, industry_emb, Wc, bc, gamma, beta, W1, as1, ad1, We1, ae1, b1, W2, as2, ad2, We2, ae2, b2):
    raise NotImplementedError("write your pallas kernel here")


# speedup vs baseline: 42.5176x; 42.5176x over previous
import functools

import jax
import jax.numpy as jnp
from jax import lax
from jax.experimental import pallas as pl
from jax.experimental.pallas import tpu as pltpu
from jax.experimental.pallas import tpu_sc as plsc

N_POL = 25000
N_COMP = 25000
N = N_POL + N_COMP
E = 800000
EMB = 32
HID = 64
HEADS = 4
C1 = HID // HEADS
OUT = 32
N_STATES = 60
N_SECTORS = 20
N_INDUSTRIES = 150
EDGE_DIM = 5

NC = 2
NS = 16
NW = NC * NS

NPAD = 51200
NPT = NPAD // NS
EPAD = 851968
TPT = EPAD // NW
NEG = -1e9

_f32 = jnp.float32


_A_BLK = EPAD // 8


def _a_body(ea_ref, we1_ref, ae1_ref, we2_ref, ae2_ref, aedge_ref, easum_ref):
    i = pl.program_id(0)
    ea = ea_ref[...]
    ae1f = (we1_ref[...].reshape(EDGE_DIM, HEADS, C1) * ae1_ref[...][None]).sum(-1)
    ae2f = (we2_ref[...] * ae2_ref[...]).sum(-1, keepdims=True)
    af = jnp.concatenate([ae1f, ae2f], axis=1)
    r = lax.dot_general(af, ea, (((0,), (0,)), ((), ())),
                        preferred_element_type=_f32)
    rows = i * _A_BLK + lax.broadcasted_iota(jnp.int32, (1, _A_BLK), 1)
    aedge_ref[...] = jnp.where(rows < E, r, NEG)

    @pl.when(i == 0)
    def _():
        easum_ref[...] = jnp.zeros_like(easum_ref)

    easum_ref[...] += ea.sum(1)


def _stage_a(eap, We1, ae1, We2, ae2):
    full = lambda s: pl.BlockSpec(s, lambda i: tuple(0 for _ in s))
    return pl.pallas_call(
        _a_body,
        grid=(8,),
        in_specs=[
            pl.BlockSpec((EDGE_DIM, _A_BLK), lambda i: (0, i)),
            full((EDGE_DIM, HID)), full((HEADS, C1)),
            full((EDGE_DIM, OUT)), full((1, OUT)),
        ],
        out_specs=[
            pl.BlockSpec((EDGE_DIM, _A_BLK), lambda i: (0, i)),
            pl.BlockSpec((EDGE_DIM,), lambda i: (0,)),
        ],
        out_shape=[
            jax.ShapeDtypeStruct((EDGE_DIM, EPAD), _f32),
            jax.ShapeDtypeStruct((EDGE_DIM,), _f32),
        ],
    )(eap, We1, ae1, We2, ae2)


_B_BLK = 1000
_B_GRID = N_POL // _B_BLK


def _lrelu(x):
    return jnp.where(x >= 0, x, 0.2 * x)


def _b_one_side(xn, w1_ref, as1_ref, ad1_ref, aem):
    h = xn @ w1_ref[...]
    hr = h.reshape(-1, HEADS, C1)
    a_s = (hr * as1_ref[...][None]).sum(-1)
    a_d = (hr * ad1_ref[...][None]).sum(-1)
    sl = jnp.exp(_lrelu(a_s + a_d + aem[None]))
    return h, a_s, a_d, sl


def _layernorm(x, g, b):
    mu = x.mean(-1, keepdims=True)
    xc = x - mu
    var = (xc * xc).mean(-1, keepdims=True)
    return xc * lax.rsqrt(var + 1e-5) * g + b


def _b_body(pf_ref, sid_ref, cf_ref, wp_ref, bp_ref, semb_ref, secemb_ref,
            indemb_ref, wc_ref, bc_ref, g_ref, be_ref, w1_ref, as1_ref,
            ad1_ref, easum_ref, we1_ref, ae1_ref,
            hp_ref, ap_ref, adp_ref, slp_ref, hc_ref, ac_ref, adc_ref, slc_ref):
    g = g_ref[...]
    be = be_ref[...]
    ae1f = (we1_ref[...].reshape(EDGE_DIM, HEADS, C1) * ae1_ref[...][None]).sum(-1)
    aem = (easum_ref[...] / float(E)) @ ae1f

    sid = sid_ref[0, 0, :]
    oh_s = (sid[:, None] == lax.broadcasted_iota(jnp.int32, (_B_BLK, N_STATES), 1)
            ).astype(_f32)
    pol = jax.nn.relu(pf_ref[...] @ wp_ref[...] + bp_ref[...]) + oh_s @ semb_ref[...]
    hp, ap, adp, slp = _b_one_side(_layernorm(pol, g, be), w1_ref, as1_ref,
                                   ad1_ref, aem)
    hp_ref[...] = hp
    ap_ref[...] = ap
    adp_ref[...] = adp
    slp_ref[...] = slp

    cf = cf_ref[...]
    sct = cf[:, 0].astype(jnp.int32)
    ind = cf[:, 1].astype(jnp.int32)
    oh_sec = (sct[:, None] == lax.broadcasted_iota(jnp.int32, (_B_BLK, N_SECTORS), 1)
              ).astype(_f32)
    oh_ind = (ind[:, None] == lax.broadcasted_iota(
        jnp.int32, (_B_BLK, N_INDUSTRIES), 1)).astype(_f32)
    wc = wc_ref[...]
    sw = secemb_ref[...] @ wc[0:8, :]
    iw = indemb_ref[...] @ wc[8:16, :]
    comp = jax.nn.relu(oh_sec @ sw + oh_ind @ iw + cf[:, 2:3] * wc[16:17, :]
                       + bc_ref[...])
    hc, ac, adc, slc = _b_one_side(_layernorm(comp, g, be), w1_ref, as1_ref,
                                   ad1_ref, aem)
    hc_ref[...] = hc
    ac_ref[...] = ac
    adc_ref[...] = adc
    slc_ref[...] = slc


def _stage_b(pf, sid3, cf, Wp, bp, state_emb, sector_emb, industry_emb, Wc, bc,
             gamma, beta, W1, as1, ad1, easum, We1, ae1):
    full = lambda s: pl.BlockSpec(s, lambda i: tuple(0 for _ in s))
    blk = lambda *s: pl.BlockSpec(s, lambda i: (i,) + tuple(0 for _ in s[1:]))
    return pl.pallas_call(
        _b_body,
        grid=(_B_GRID,),
        in_specs=[
            blk(_B_BLK, 7),
            pl.BlockSpec((1, 1, _B_BLK), lambda i: (i, 0, 0)),
            blk(_B_BLK, 3),
            full((7, EMB)), full((EMB,)), full((N_STATES, EMB)),
            full((N_SECTORS, 8)), full((N_INDUSTRIES, 8)), full((17, EMB)),
            full((EMB,)), full((EMB,)), full((EMB,)), full((EMB, HID)),
            full((HEADS, C1)), full((HEADS, C1)), full((EDGE_DIM,)),
            full((EDGE_DIM, HID)), full((HEADS, C1)),
        ],
        out_specs=[
            blk(_B_BLK, HID), blk(_B_BLK, HEADS), blk(_B_BLK, HEADS),
            blk(_B_BLK, HEADS),
            blk(_B_BLK, HID), blk(_B_BLK, HEADS), blk(_B_BLK, HEADS),
            blk(_B_BLK, HEADS),
        ],
        out_shape=[
            jax.ShapeDtypeStruct((N_POL, HID), _f32),
            jax.ShapeDtypeStruct((N_POL, HEADS), _f32),
            jax.ShapeDtypeStruct((N_POL, HEADS), _f32),
            jax.ShapeDtypeStruct((N_POL, HEADS), _f32),
            jax.ShapeDtypeStruct((N_COMP, HID), _f32),
            jax.ShapeDtypeStruct((N_COMP, HEADS), _f32),
            jax.ShapeDtypeStruct((N_COMP, HEADS), _f32),
            jax.ShapeDtypeStruct((N_COMP, HEADS), _f32),
        ],
    )(pf, sid3, cf, Wp, bp, state_emb, sector_emb, industry_emb, Wc, bc,
      gamma, beta, W1, as1, ad1, easum, We1, ae1)


def _make_sc_pass(D, CH):
    K = CH // 128
    NCHUNK = TPT // CH
    ND16 = D // 16
    mesh = plsc.VectorSubcoreMesh(core_axis_name="c", subcore_axis_name="s",
                                  num_cores=NC, num_subcores=NS)

    def body(htab, asrc, adst, srcm, dstm, aeh, s_out, u_out,
             src_v, dst_v, ae_v, ex_v, asg_v, adg_v, rows_v, zrow_v, zs_v,
             sh_as, sh_ad, s_acc, u_acc, sem, sem2):
        cid = lax.axis_index("c")
        sid = lax.axis_index("s")
        wid = sid * NC + cid
        nb = sid * NPT

        pltpu.sync_copy(asrc.at[pl.ds(nb, NPT)], sh_as.at[pl.ds(nb, NPT)])
        pltpu.sync_copy(adst.at[pl.ds(nb, NPT)], sh_ad.at[pl.ds(nb, NPT)])

        z16 = jnp.zeros((16,), _f32)

        def zr(i, _):
            for t in range(ND16):
                zrow_v[i, pl.ds(t * 16, 16)] = z16
            return 0

        lax.fori_loop(0, NPT // 16, zr, 0)

        def zs(i, _):
            zs_v[pl.ds(i * 16, 16)] = z16
            return 0

        lax.fori_loop(0, 400 // 16, zs, 0)

        for t in range(NS):
            pltpu.sync_copy(zrow_v, u_acc.at[pl.ds(nb + t * (NPT // 16),
                                                   NPT // 16)])
        for t in range(8):
            pltpu.sync_copy(zs_v, s_acc.at[pl.ds(nb + t * 400, 400)])
        plsc.subcore_barrier()

        def chunk(c, _):
            rb = wid * (TPT // 128) + c * K
            eb = wid * TPT + c * CH
            pltpu.sync_copy(srcm.at[pl.ds(rb, K)], src_v)
            pltpu.sync_copy(dstm.at[pl.ds(rb, K)], dst_v)
            pltpu.sync_copy(aeh.at[pl.ds(eb, CH)], ae_v)

            descs = [pltpu.async_copy(htab.at[src_v.at[k]],
                                      rows_v.at[pl.ds(k * 128, 128)], sem)
                     for k in range(K)]
            descs += [pltpu.async_copy(sh_as.at[src_v.at[k]],
                                       asg_v.at[pl.ds(k * 128, 128)], sem2)
                      for k in range(K)]
            descs += [pltpu.async_copy(sh_ad.at[dst_v.at[k]],
                                       adg_v.at[pl.ds(k * 128, 128)], sem2)
                      for k in range(K)]
            for d in descs:
                d.wait()

            def exl(l, _):
                o = l * 16
                a = asg_v[pl.ds(o, 16)] + adg_v[pl.ds(o, 16)] + ae_v[pl.ds(o, 16)]
                a = jnp.where(a >= 0, a, a * 0.2)
                ex_v[pl.ds(o, 16)] = jnp.exp(a)
                return 0

            lax.fori_loop(0, CH // 16, exl, 0)

            def scl(g, _):
                ex16 = ex_v[pl.ds(g * 16, 16)]
                for l in range(16):
                    e = ex16[l]
                    j = g * 16 + l
                    for t in range(ND16):
                        rows_v[j, pl.ds(t * 16, 16)] = (
                            rows_v[j, pl.ds(t * 16, 16)] * e)
                return 0

            lax.fori_loop(0, CH // 16, scl, 0)

            for k in range(K):
                pltpu.sync_copy(ex_v.at[pl.ds(k * 128, 128)],
                                s_acc.at[dst_v.at[k]], add=True)
                pltpu.sync_copy(rows_v.at[pl.ds(k * 128, 128)],
                                u_acc.at[dst_v.at[k]], add=True)
            return 0

        lax.fori_loop(0, NCHUNK, chunk, 0)
        plsc.subcore_barrier()

        pltpu.sync_copy(s_acc.at[pl.ds(nb, NPT)],
                        s_out.at[pl.ds(cid * NPAD + nb, NPT)])
        pltpu.sync_copy(u_acc.at[pl.ds(nb, NPT)], u_out.at[cid, pl.ds(nb, NPT)])

    return pl.kernel(
        body,
        out_type=[
            jax.ShapeDtypeStruct((NC * NPAD,), _f32),
            jax.ShapeDtypeStruct((NC, NPAD, D), _f32),
        ],
        mesh=mesh,
        compiler_params=pltpu.CompilerParams(needs_layout_passes=False,
                                             use_tc_tiling_on_sc=False),
        scratch_types=[
            pltpu.VMEM((K, 128), jnp.int32),
            pltpu.VMEM((K, 128), jnp.int32),
            pltpu.VMEM((CH,), _f32),
            pltpu.VMEM((CH,), _f32),
            pltpu.VMEM((CH,), _f32),
            pltpu.VMEM((CH,), _f32),
            pltpu.VMEM((CH, D), _f32),
            pltpu.VMEM((NPT // 16, D), _f32),
            pltpu.VMEM((400,), _f32),
            pltpu.VMEM_SHARED((NPAD,), _f32),
            pltpu.VMEM_SHARED((NPAD,), _f32),
            pltpu.VMEM_SHARED((NPAD,), _f32),
            pltpu.VMEM_SHARED((NPAD, D), _f32),
            pltpu.SemaphoreType.DMA,
            pltpu.SemaphoreType.DMA,
        ],
    )


_C_BLK = 1024
_C_GRID = NPAD // _C_BLK


def _c_body(s1p_ref, u1p_ref, sl1_ref, h1_ref, b1_ref, w2_ref, as2_ref,
            ad2_ref, easum_ref, we2_ref, ae2_ref,
            h2a_ref, h2b_ref, a2_ref, ad2o_ref, sl2_ref):
    h1 = h1_ref[...]
    outs = []
    for h in range(HEADS):
        slh = sl1_ref[h]
        uh = (u1p_ref[2 * h] + u1p_ref[2 * h + 1]
              + slh[:, None] * h1[:, h * C1:(h + 1) * C1])
        sh = s1p_ref[2 * h] + s1p_ref[2 * h + 1] + slh
        outs.append(uh / (sh[:, None] + 1e-16))
    out1 = jnp.concatenate(outs, axis=-1) + b1_ref[...]
    x2 = jnp.where(out1 > 0, out1, jnp.exp(jnp.minimum(out1, 0.0)) - 1.0)
    h2 = x2 @ w2_ref[...]
    a2 = (h2 * as2_ref[...]).sum(-1)
    ad2v = (h2 * ad2_ref[...]).sum(-1)
    ae2f = (we2_ref[...] * ae2_ref[...]).sum(-1)
    aem2 = ((easum_ref[...] / float(E)) * ae2f).sum()
    h2a_ref[...] = h2[:, :C1]
    h2b_ref[...] = h2[:, C1:]
    a2_ref[...] = a2
    ad2o_ref[...] = ad2v
    sl2_ref[...] = jnp.exp(_lrelu(a2 + ad2v + aem2))


def _stage_c(s1p, u1p, sl1, h1p, b1, W2, as2, ad2, easum, We2, ae2):
    full = lambda s: pl.BlockSpec(s, lambda i: tuple(0 for _ in s))
    return pl.pallas_call(
        _c_body,
        grid=(_C_GRID,),
        in_specs=[
            pl.BlockSpec((2 * HEADS, _C_BLK), lambda i: (0, i)),
            pl.BlockSpec((2 * HEADS, _C_BLK, C1), lambda i: (0, i, 0)),
            pl.BlockSpec((HEADS, _C_BLK), lambda i: (0, i)),
            pl.BlockSpec((_C_BLK, HID), lambda i: (i, 0)),
            full((HID,)), full((HID, OUT)), full((1, OUT)), full((1, OUT)),
            full((EDGE_DIM,)), full((EDGE_DIM, OUT)), full((1, OUT)),
        ],
        out_specs=[
            pl.BlockSpec((_C_BLK, C1), lambda i: (i, 0)),
            pl.BlockSpec((_C_BLK, C1), lambda i: (i, 0)),
            pl.BlockSpec((_C_BLK,), lambda i: (i,)),
            pl.BlockSpec((_C_BLK,), lambda i: (i,)),
            pl.BlockSpec((_C_BLK,), lambda i: (i,)),
        ],
        out_shape=[
            jax.ShapeDtypeStruct((NPAD, C1), _f32),
            jax.ShapeDtypeStruct((NPAD, C1), _f32),
            jax.ShapeDtypeStruct((NPAD,), _f32),
            jax.ShapeDtypeStruct((NPAD,), _f32),
            jax.ShapeDtypeStruct((NPAD,), _f32),
        ],
    )(s1p, u1p, sl1, h1p, b1, W2, as2, ad2, easum, We2, ae2)


def _d_body(s2p_ref, u2pa_ref, u2pb_ref, sl2_ref, h2a_ref, h2b_ref, b2_ref,
            out_ref):
    sl2 = sl2_ref[...]
    s = s2p_ref[0] + s2p_ref[1] + sl2
    ua = u2pa_ref[0] + u2pa_ref[1] + sl2[:, None] * h2a_ref[...]
    ub = u2pb_ref[0] + u2pb_ref[1] + sl2[:, None] * h2b_ref[...]
    u = jnp.concatenate([ua, ub], axis=-1)
    out_ref[...] = u / (s[:, None] + 1e-16) + b2_ref[...]


def _stage_d(s2p, u2pa, u2pb, sl2, h2a, h2b, b2):
    full = lambda s: pl.BlockSpec(s, lambda i: tuple(0 for _ in s))
    return pl.pallas_call(
        _d_body,
        grid=(_C_GRID,),
        in_specs=[
            pl.BlockSpec((2, _C_BLK), lambda i: (0, i)),
            pl.BlockSpec((2, _C_BLK, C1), lambda i: (0, i, 0)),
            pl.BlockSpec((2, _C_BLK, C1), lambda i: (0, i, 0)),
            pl.BlockSpec((_C_BLK,), lambda i: (i,)),
            pl.BlockSpec((_C_BLK, C1), lambda i: (i, 0)),
            pl.BlockSpec((_C_BLK, C1), lambda i: (i, 0)),
            full((OUT,)),
        ],
        out_specs=pl.BlockSpec((_C_BLK, OUT), lambda i: (i, 0)),
        out_shape=jax.ShapeDtypeStruct((NPAD, OUT), _f32),
    )(s2p, u2pa, u2pb, sl2, h2a, h2b, b2)


@jax.jit
def kernel(edge_index, edge_attr, pol_features, state_ids, comp_features, Wp,
           bp, state_emb, sector_emb, industry_emb, Wc, bc, gamma, beta, W1,
           as1, ad1, We1, ae1, b1, W2, as2, ad2, We2, ae2, b2):
    src = edge_index[0]
    dst = edge_index[1]
    zi = jnp.zeros((EPAD - E,), jnp.int32)
    srcm = jnp.concatenate([src, zi]).reshape(EPAD // 128, 128)
    dstm = jnp.concatenate([dst, zi]).reshape(EPAD // 128, 128)
    eap = jnp.concatenate([edge_attr, jnp.zeros((EPAD - E, EDGE_DIM), _f32)],
                          0).T

    aedge, easum = _stage_a(eap, We1, ae1, We2, ae2)

    sid3 = state_ids.reshape(_B_GRID, 1, _B_BLK)
    hp, ap, adp, slp, hc, ac, adc, slc = _stage_b(
        pol_features, sid3, comp_features, Wp, bp, state_emb, sector_emb,
        industry_emb, Wc, bc, gamma, beta, W1, as1, ad1, easum, We1, ae1)

    pad_n = ((0, NPAD - N), (0, 0))
    h1p = jnp.pad(jnp.concatenate([hp, hc], 0), pad_n)
    asrc1 = jnp.pad(jnp.concatenate([ap, ac], 0), pad_n).T
    adst1 = jnp.pad(jnp.concatenate([adp, adc], 0), pad_n).T
    sl1 = jnp.pad(jnp.concatenate([slp, slc], 0), pad_n).T
    htab1 = h1p.reshape(NPAD, HEADS, C1).transpose(1, 0, 2)

    sc1 = _make_sc_pass(C1, 1024)
    s_parts, u_parts = [], []
    for h in range(HEADS):
        s_h, u_h = sc1(htab1[h], asrc1[h], adst1[h], srcm, dstm, aedge[h])
        s_parts.append(s_h.reshape(NC, NPAD))
        u_parts.append(u_h)
    s1p = jnp.stack(s_parts).reshape(2 * HEADS, NPAD)
    u1p = jnp.stack(u_parts).reshape(2 * HEADS, NPAD, C1)

    h2a, h2b, asrc2, adst2, sl2 = _stage_c(s1p, u1p, sl1, h1p, b1, W2, as2,
                                           ad2, easum, We2, ae2)

    s2p, u2pa = sc1(h2a, asrc2, adst2, srcm, dstm, aedge[HEADS])
    _, u2pb = sc1(h2b, asrc2, adst2, srcm, dstm, aedge[HEADS])
    s2p = s2p.reshape(NC, NPAD)

    out = _stage_d(s2p, u2pa, u2pb, sl2, h2a, h2b, b2)
    return out[:N]


# speedup vs baseline: 45.3165x; 1.0658x over previous
import functools

import jax
import jax.numpy as jnp
from jax import lax
from jax.experimental import pallas as pl
from jax.experimental.pallas import tpu as pltpu
from jax.experimental.pallas import tpu_sc as plsc

N_POL = 25000
N_COMP = 25000
N = N_POL + N_COMP
E = 800000
EMB = 32
HID = 64
HEADS = 4
C1 = HID // HEADS
OUT = 32
N_STATES = 60
N_SECTORS = 20
N_INDUSTRIES = 150
EDGE_DIM = 5

NC = 2
NS = 16
NW = NC * NS

NPAD = 51200
NPT = NPAD // NS
EPAD = 851968
TPT = EPAD // NW
NEG = -1e9

_f32 = jnp.float32


_A_BLK = EPAD // 8


def _a_body(ea_ref, we1_ref, ae1_ref, we2_ref, ae2_ref, aedge_ref, easum_ref):
    i = pl.program_id(0)
    ea = ea_ref[...]
    ae1f = (we1_ref[...].reshape(EDGE_DIM, HEADS, C1) * ae1_ref[...][None]).sum(-1)
    ae2f = (we2_ref[...] * ae2_ref[...]).sum(-1, keepdims=True)
    af = jnp.concatenate([ae1f, ae2f], axis=1)
    r = lax.dot_general(af, ea, (((0,), (0,)), ((), ())),
                        preferred_element_type=_f32)
    rows = i * _A_BLK + lax.broadcasted_iota(jnp.int32, (1, _A_BLK), 1)
    aedge_ref[...] = jnp.where(rows < E, r, NEG)

    @pl.when(i == 0)
    def _():
        easum_ref[...] = jnp.zeros_like(easum_ref)

    easum_ref[...] += ea.sum(1)


def _stage_a(eap, We1, ae1, We2, ae2):
    full = lambda s: pl.BlockSpec(s, lambda i: tuple(0 for _ in s))
    return pl.pallas_call(
        _a_body,
        grid=(8,),
        in_specs=[
            pl.BlockSpec((EDGE_DIM, _A_BLK), lambda i: (0, i)),
            full((EDGE_DIM, HID)), full((HEADS, C1)),
            full((EDGE_DIM, OUT)), full((1, OUT)),
        ],
        out_specs=[
            pl.BlockSpec((EDGE_DIM, _A_BLK), lambda i: (0, i)),
            pl.BlockSpec((EDGE_DIM,), lambda i: (0,)),
        ],
        out_shape=[
            jax.ShapeDtypeStruct((EDGE_DIM, EPAD), _f32),
            jax.ShapeDtypeStruct((EDGE_DIM,), _f32),
        ],
    )(eap, We1, ae1, We2, ae2)


_B_BLK = 1000
_B_GRID = N_POL // _B_BLK


def _lrelu(x):
    return jnp.where(x >= 0, x, 0.2 * x)


def _b_one_side(xn, w1_ref, as1_ref, ad1_ref, aem):
    h = xn @ w1_ref[...]
    hr = h.reshape(-1, HEADS, C1)
    a_s = (hr * as1_ref[...][None]).sum(-1)
    a_d = (hr * ad1_ref[...][None]).sum(-1)
    sl = jnp.exp(_lrelu(a_s + a_d + aem[None]))
    return h, a_s, a_d, sl


def _layernorm(x, g, b):
    mu = x.mean(-1, keepdims=True)
    xc = x - mu
    var = (xc * xc).mean(-1, keepdims=True)
    return xc * lax.rsqrt(var + 1e-5) * g + b


def _b_body(pf_ref, sid_ref, cf_ref, wp_ref, bp_ref, semb_ref, secemb_ref,
            indemb_ref, wc_ref, bc_ref, g_ref, be_ref, w1_ref, as1_ref,
            ad1_ref, easum_ref, we1_ref, ae1_ref,
            hp_ref, ap_ref, adp_ref, slp_ref, hc_ref, ac_ref, adc_ref, slc_ref):
    g = g_ref[...]
    be = be_ref[...]
    ae1f = (we1_ref[...].reshape(EDGE_DIM, HEADS, C1) * ae1_ref[...][None]).sum(-1)
    aem = (easum_ref[...] / float(E)) @ ae1f

    sid = sid_ref[0, 0, :]
    oh_s = (sid[:, None] == lax.broadcasted_iota(jnp.int32, (_B_BLK, N_STATES), 1)
            ).astype(_f32)
    pol = jax.nn.relu(pf_ref[...] @ wp_ref[...] + bp_ref[...]) + oh_s @ semb_ref[...]
    hp, ap, adp, slp = _b_one_side(_layernorm(pol, g, be), w1_ref, as1_ref,
                                   ad1_ref, aem)
    hp_ref[...] = hp
    ap_ref[...] = ap
    adp_ref[...] = adp
    slp_ref[...] = slp

    cf = cf_ref[...]
    sct = cf[:, 0].astype(jnp.int32)
    ind = cf[:, 1].astype(jnp.int32)
    oh_sec = (sct[:, None] == lax.broadcasted_iota(jnp.int32, (_B_BLK, N_SECTORS), 1)
              ).astype(_f32)
    oh_ind = (ind[:, None] == lax.broadcasted_iota(
        jnp.int32, (_B_BLK, N_INDUSTRIES), 1)).astype(_f32)
    wc = wc_ref[...]
    sw = secemb_ref[...] @ wc[0:8, :]
    iw = indemb_ref[...] @ wc[8:16, :]
    comp = jax.nn.relu(oh_sec @ sw + oh_ind @ iw + cf[:, 2:3] * wc[16:17, :]
                       + bc_ref[...])
    hc, ac, adc, slc = _b_one_side(_layernorm(comp, g, be), w1_ref, as1_ref,
                                   ad1_ref, aem)
    hc_ref[...] = hc
    ac_ref[...] = ac
    adc_ref[...] = adc
    slc_ref[...] = slc


def _stage_b(pf, sid3, cf, Wp, bp, state_emb, sector_emb, industry_emb, Wc, bc,
             gamma, beta, W1, as1, ad1, easum, We1, ae1):
    full = lambda s: pl.BlockSpec(s, lambda i: tuple(0 for _ in s))
    blk = lambda *s: pl.BlockSpec(s, lambda i: (i,) + tuple(0 for _ in s[1:]))
    return pl.pallas_call(
        _b_body,
        grid=(_B_GRID,),
        in_specs=[
            blk(_B_BLK, 7),
            pl.BlockSpec((1, 1, _B_BLK), lambda i: (i, 0, 0)),
            blk(_B_BLK, 3),
            full((7, EMB)), full((EMB,)), full((N_STATES, EMB)),
            full((N_SECTORS, 8)), full((N_INDUSTRIES, 8)), full((17, EMB)),
            full((EMB,)), full((EMB,)), full((EMB,)), full((EMB, HID)),
            full((HEADS, C1)), full((HEADS, C1)), full((EDGE_DIM,)),
            full((EDGE_DIM, HID)), full((HEADS, C1)),
        ],
        out_specs=[
            blk(_B_BLK, HID), blk(_B_BLK, HEADS), blk(_B_BLK, HEADS),
            blk(_B_BLK, HEADS),
            blk(_B_BLK, HID), blk(_B_BLK, HEADS), blk(_B_BLK, HEADS),
            blk(_B_BLK, HEADS),
        ],
        out_shape=[
            jax.ShapeDtypeStruct((N_POL, HID), _f32),
            jax.ShapeDtypeStruct((N_POL, HEADS), _f32),
            jax.ShapeDtypeStruct((N_POL, HEADS), _f32),
            jax.ShapeDtypeStruct((N_POL, HEADS), _f32),
            jax.ShapeDtypeStruct((N_COMP, HID), _f32),
            jax.ShapeDtypeStruct((N_COMP, HEADS), _f32),
            jax.ShapeDtypeStruct((N_COMP, HEADS), _f32),
            jax.ShapeDtypeStruct((N_COMP, HEADS), _f32),
        ],
    )(pf, sid3, cf, Wp, bp, state_emb, sector_emb, industry_emb, Wc, bc,
      gamma, beta, W1, as1, ad1, easum, We1, ae1)


def _make_sc_pass(D, CH):
    K = CH // 128
    NCHUNK = TPT // CH
    mesh = plsc.VectorSubcoreMesh(core_axis_name="c", subcore_axis_name="s",
                                  num_cores=NC, num_subcores=NS)

    def body(htab, asrc, adst, srcm, dstm, aeh, s_out, u_out,
             src_v, dst_v, ae_v, ex_v, asg_v, adg_v, rows_v, zrow_v, zs_v,
             sh_as, sh_ad, s_acc, u_acc, semL, semG, semG2, semS):
        cid = lax.axis_index("c")
        sid = lax.axis_index("s")
        wid = sid * NC + cid
        nb = sid * NPT

        pltpu.sync_copy(asrc.at[pl.ds(nb, NPT)], sh_as.at[pl.ds(nb, NPT)])
        pltpu.sync_copy(adst.at[pl.ds(nb, NPT)], sh_ad.at[pl.ds(nb, NPT)])

        z16 = jnp.zeros((16,), _f32)

        def zr(i, _):
            zrow_v[i, pl.ds(0, 16)] = z16
            return 0

        lax.fori_loop(0, NPT // 16, zr, 0)

        def zs(i, _):
            zs_v[pl.ds(i * 16, 16)] = z16
            return 0

        lax.fori_loop(0, 400 // 16, zs, 0)

        for t in range(NS):
            pltpu.sync_copy(zrow_v, u_acc.at[pl.ds(nb + t * (NPT // 16),
                                                   NPT // 16)])
        for t in range(8):
            pltpu.sync_copy(zs_v, s_acc.at[pl.ds(nb + t * 400, 400)])
        plsc.subcore_barrier()

        def pair(i, _):
            ld = []
            for b in range(2):
                c = 2 * i + b
                rb = wid * (TPT // 128) + c * K
                eb = wid * TPT + c * CH
                ld.append([
                    pltpu.async_copy(srcm.at[pl.ds(rb, K)],
                                     src_v.at[pl.ds(b * K, K)], semL),
                    pltpu.async_copy(dstm.at[pl.ds(rb, K)],
                                     dst_v.at[pl.ds(b * K, K)], semL),
                    pltpu.async_copy(aeh.at[pl.ds(eb, CH)],
                                     ae_v.at[pl.ds(b * CH, CH)], semL),
                ])
            g = []
            for b in range(2):
                for d in ld[b]:
                    d.wait()
                gh = [pltpu.async_copy(htab.at[src_v.at[b * K + k]],
                                       rows_v.at[pl.ds(b * CH + k * 128, 128)],
                                       semG) for k in range(K)]
                gh += [pltpu.async_copy(sh_as.at[src_v.at[b * K + k]],
                                        asg_v.at[pl.ds(b * CH + k * 128, 128)],
                                        semG2) for k in range(K)]
                gh += [pltpu.async_copy(sh_ad.at[dst_v.at[b * K + k]],
                                        adg_v.at[pl.ds(b * CH + k * 128, 128)],
                                        semG2) for k in range(K)]
                g.append(gh)
            sd = []
            for b in range(2):
                for d in g[b]:
                    d.wait()
                ob = b * CH

                def exl(l, _, ob=ob):
                    o = ob + l * 16
                    a = (asg_v[pl.ds(o, 16)] + adg_v[pl.ds(o, 16)]
                         + ae_v[pl.ds(o, 16)])
                    a = jnp.where(a >= 0, a, a * 0.2)
                    ex_v[pl.ds(o, 16)] = jnp.exp(a)
                    return 0

                lax.fori_loop(0, CH // 16, exl, 0)

                def scl(gi, _, ob=ob):
                    ex16 = ex_v[pl.ds(ob + gi * 16, 16)]
                    for l in range(16):
                        e = ex16[l]
                        j = ob + gi * 16 + l
                        rows_v[j, pl.ds(0, 16)] = rows_v[j, pl.ds(0, 16)] * e
                    return 0

                lax.fori_loop(0, CH // 16, scl, 0)

                for k in range(K):
                    sd.append(pltpu.async_copy(
                        ex_v.at[pl.ds(ob + k * 128, 128)],
                        s_acc.at[dst_v.at[b * K + k]], semS, add=True))
                    sd.append(pltpu.async_copy(
                        rows_v.at[pl.ds(ob + k * 128, 128)],
                        u_acc.at[dst_v.at[b * K + k]], semS, add=True))
            for d in sd:
                d.wait()
            return 0

        lax.fori_loop(0, NCHUNK // 2, pair, 0)
        plsc.subcore_barrier()

        pltpu.sync_copy(s_acc.at[pl.ds(nb, NPT)],
                        s_out.at[pl.ds(cid * NPAD + nb, NPT)])
        pltpu.sync_copy(u_acc.at[pl.ds(nb, NPT)], u_out.at[cid, pl.ds(nb, NPT)])

    return pl.kernel(
        body,
        out_type=[
            jax.ShapeDtypeStruct((NC * NPAD,), _f32),
            jax.ShapeDtypeStruct((NC, NPAD, D), _f32),
        ],
        mesh=mesh,
        compiler_params=pltpu.CompilerParams(needs_layout_passes=False,
                                             use_tc_tiling_on_sc=False),
        scratch_types=[
            pltpu.VMEM((2 * K, 128), jnp.int32),
            pltpu.VMEM((2 * K, 128), jnp.int32),
            pltpu.VMEM((2 * CH,), _f32),
            pltpu.VMEM((2 * CH,), _f32),
            pltpu.VMEM((2 * CH,), _f32),
            pltpu.VMEM((2 * CH,), _f32),
            pltpu.VMEM((2 * CH, D), _f32),
            pltpu.VMEM((NPT // 16, D), _f32),
            pltpu.VMEM((400,), _f32),
            pltpu.VMEM_SHARED((NPAD,), _f32),
            pltpu.VMEM_SHARED((NPAD,), _f32),
            pltpu.VMEM_SHARED((NPAD,), _f32),
            pltpu.VMEM_SHARED((NPAD, D), _f32),
            pltpu.SemaphoreType.DMA,
            pltpu.SemaphoreType.DMA,
            pltpu.SemaphoreType.DMA,
            pltpu.SemaphoreType.DMA,
        ],
    )


_C_BLK = 1024
_C_GRID = NPAD // _C_BLK


def _c_body(s1p_ref, u1p_ref, sl1_ref, h1_ref, b1_ref, w2_ref, as2_ref,
            ad2_ref, easum_ref, we2_ref, ae2_ref,
            h2a_ref, h2b_ref, a2_ref, ad2o_ref, sl2_ref):
    h1 = h1_ref[...]
    outs = []
    for h in range(HEADS):
        slh = sl1_ref[h]
        uh = (u1p_ref[2 * h] + u1p_ref[2 * h + 1]
              + slh[:, None] * h1[:, h * C1:(h + 1) * C1])
        sh = s1p_ref[2 * h] + s1p_ref[2 * h + 1] + slh
        outs.append(uh / (sh[:, None] + 1e-16))
    out1 = jnp.concatenate(outs, axis=-1) + b1_ref[...]
    x2 = jnp.where(out1 > 0, out1, jnp.exp(jnp.minimum(out1, 0.0)) - 1.0)
    h2 = x2 @ w2_ref[...]
    a2 = (h2 * as2_ref[...]).sum(-1)
    ad2v = (h2 * ad2_ref[...]).sum(-1)
    ae2f = (we2_ref[...] * ae2_ref[...]).sum(-1)
    aem2 = ((easum_ref[...] / float(E)) * ae2f).sum()
    h2a_ref[...] = h2[:, :C1]
    h2b_ref[...] = h2[:, C1:]
    a2_ref[...] = a2
    ad2o_ref[...] = ad2v
    sl2_ref[...] = jnp.exp(_lrelu(a2 + ad2v + aem2))


def _stage_c(s1p, u1p, sl1, h1p, b1, W2, as2, ad2, easum, We2, ae2):
    full = lambda s: pl.BlockSpec(s, lambda i: tuple(0 for _ in s))
    return pl.pallas_call(
        _c_body,
        grid=(_C_GRID,),
        in_specs=[
            pl.BlockSpec((2 * HEADS, _C_BLK), lambda i: (0, i)),
            pl.BlockSpec((2 * HEADS, _C_BLK, C1), lambda i: (0, i, 0)),
            pl.BlockSpec((HEADS, _C_BLK), lambda i: (0, i)),
            pl.BlockSpec((_C_BLK, HID), lambda i: (i, 0)),
            full((HID,)), full((HID, OUT)), full((1, OUT)), full((1, OUT)),
            full((EDGE_DIM,)), full((EDGE_DIM, OUT)), full((1, OUT)),
        ],
        out_specs=[
            pl.BlockSpec((_C_BLK, C1), lambda i: (i, 0)),
            pl.BlockSpec((_C_BLK, C1), lambda i: (i, 0)),
            pl.BlockSpec((_C_BLK,), lambda i: (i,)),
            pl.BlockSpec((_C_BLK,), lambda i: (i,)),
            pl.BlockSpec((_C_BLK,), lambda i: (i,)),
        ],
        out_shape=[
            jax.ShapeDtypeStruct((NPAD, C1), _f32),
            jax.ShapeDtypeStruct((NPAD, C1), _f32),
            jax.ShapeDtypeStruct((NPAD,), _f32),
            jax.ShapeDtypeStruct((NPAD,), _f32),
            jax.ShapeDtypeStruct((NPAD,), _f32),
        ],
    )(s1p, u1p, sl1, h1p, b1, W2, as2, ad2, easum, We2, ae2)


def _d_body(s2p_ref, u2pa_ref, u2pb_ref, sl2_ref, h2a_ref, h2b_ref, b2_ref,
            out_ref):
    sl2 = sl2_ref[...]
    s = s2p_ref[0] + s2p_ref[1] + sl2
    ua = u2pa_ref[0] + u2pa_ref[1] + sl2[:, None] * h2a_ref[...]
    ub = u2pb_ref[0] + u2pb_ref[1] + sl2[:, None] * h2b_ref[...]
    u = jnp.concatenate([ua, ub], axis=-1)
    out_ref[...] = u / (s[:, None] + 1e-16) + b2_ref[...]


def _stage_d(s2p, u2pa, u2pb, sl2, h2a, h2b, b2):
    full = lambda s: pl.BlockSpec(s, lambda i: tuple(0 for _ in s))
    return pl.pallas_call(
        _d_body,
        grid=(_C_GRID,),
        in_specs=[
            pl.BlockSpec((2, _C_BLK), lambda i: (0, i)),
            pl.BlockSpec((2, _C_BLK, C1), lambda i: (0, i, 0)),
            pl.BlockSpec((2, _C_BLK, C1), lambda i: (0, i, 0)),
            pl.BlockSpec((_C_BLK,), lambda i: (i,)),
            pl.BlockSpec((_C_BLK, C1), lambda i: (i, 0)),
            pl.BlockSpec((_C_BLK, C1), lambda i: (i, 0)),
            full((OUT,)),
        ],
        out_specs=pl.BlockSpec((_C_BLK, OUT), lambda i: (i, 0)),
        out_shape=jax.ShapeDtypeStruct((NPAD, OUT), _f32),
    )(s2p, u2pa, u2pb, sl2, h2a, h2b, b2)


@jax.jit
def kernel(edge_index, edge_attr, pol_features, state_ids, comp_features, Wp,
           bp, state_emb, sector_emb, industry_emb, Wc, bc, gamma, beta, W1,
           as1, ad1, We1, ae1, b1, W2, as2, ad2, We2, ae2, b2):
    src = edge_index[0]
    dst = edge_index[1]
    zi = jnp.zeros((EPAD - E,), jnp.int32)
    srcm = jnp.concatenate([src, zi]).reshape(EPAD // 128, 128)
    dstm = jnp.concatenate([dst, zi]).reshape(EPAD // 128, 128)
    eap = jnp.concatenate([edge_attr, jnp.zeros((EPAD - E, EDGE_DIM), _f32)],
                          0).T

    aedge, easum = _stage_a(eap, We1, ae1, We2, ae2)

    sid3 = state_ids.reshape(_B_GRID, 1, _B_BLK)
    hp, ap, adp, slp, hc, ac, adc, slc = _stage_b(
        pol_features, sid3, comp_features, Wp, bp, state_emb, sector_emb,
        industry_emb, Wc, bc, gamma, beta, W1, as1, ad1, easum, We1, ae1)

    pad_n = ((0, NPAD - N), (0, 0))
    h1p = jnp.pad(jnp.concatenate([hp, hc], 0), pad_n)
    asrc1 = jnp.pad(jnp.concatenate([ap, ac], 0), pad_n).T
    adst1 = jnp.pad(jnp.concatenate([adp, adc], 0), pad_n).T
    sl1 = jnp.pad(jnp.concatenate([slp, slc], 0), pad_n).T
    htab1 = h1p.reshape(NPAD, HEADS, C1).transpose(1, 0, 2)

    sc1 = _make_sc_pass(C1, 1024)
    s_parts, u_parts = [], []
    for h in range(HEADS):
        s_h, u_h = sc1(htab1[h], asrc1[h], adst1[h], srcm, dstm, aedge[h])
        s_parts.append(s_h.reshape(NC, NPAD))
        u_parts.append(u_h)
    s1p = jnp.stack(s_parts).reshape(2 * HEADS, NPAD)
    u1p = jnp.stack(u_parts).reshape(2 * HEADS, NPAD, C1)

    h2a, h2b, asrc2, adst2, sl2 = _stage_c(s1p, u1p, sl1, h1p, b1, W2, as2,
                                           ad2, easum, We2, ae2)

    s2p, u2pa = sc1(h2a, asrc2, adst2, srcm, dstm, aedge[HEADS])
    _, u2pb = sc1(h2b, asrc2, adst2, srcm, dstm, aedge[HEADS])
    s2p = s2p.reshape(NC, NPAD)

    out = _stage_d(s2p, u2pa, u2pb, sl2, h2a, h2b, b2)
    return out[:N]
